# baseline (speedup 1.0000x reference)
import functools

import jax
import jax.numpy as jnp
from jax import lax
from jax.experimental import pallas as pl
from jax.experimental.pallas import tpu as pltpu
from jax.experimental.pallas import tpu_sc as plsc

B = 8
N = 4096
S = 512
K = 32
M = B * S * K
RADIUS_SQ = 1.0
NTILES = 32
SPT = S // (NTILES // B)
TPB = NTILES // B
CPT = SPT * K


def _fps_body(cp_ref, far0_ref, scout_ref, ssq_ref):
    x = cp_ref[0]
    y = cp_ref[1]
    z = cp_ref[2]
    ssq_ref[...] = (x * x + y * y) + z * z
    iota = lax.broadcasted_iota(jnp.int32, (B, N), 1)
    iota_s = lax.broadcasted_iota(jnp.int32, (B, S), 1)

    def body(i, carry):
        dist, far, sx, sy, sz = carry
        oh = iota == far
        cxs = jnp.sum(jnp.where(oh, x, 0.0), axis=1, keepdims=True)
        cys = jnp.sum(jnp.where(oh, y, 0.0), axis=1, keepdims=True)
        czs = jnp.sum(jnp.where(oh, z, 0.0), axis=1, keepdims=True)
        hit = iota_s == i
        sx = jnp.where(hit, cxs, sx)
        sy = jnp.where(hit, cys, sy)
        sz = jnp.where(hit, czs, sz)
        dx = x - cxs
        dy = y - cys
        dz = z - czs
        d = (dx * dx + dy * dy) + dz * dz
        dist = jnp.minimum(dist, d)
        m = jnp.max(dist, axis=1, keepdims=True)
        far = jnp.min(jnp.where(dist == m, iota, jnp.int32(N)), axis=1,
                      keepdims=True)
        return dist, far, sx, sy, sz

    dist0 = jnp.full((B, N), jnp.inf, dtype=jnp.float32)
    z0 = jnp.zeros((B, S), dtype=jnp.float32)
    _, _, sx, sy, sz = lax.fori_loop(
        0, S, body, (dist0, far0_ref[...], z0, z0, z0))
    scout_ref[0] = sx
    scout_ref[1] = sy
    scout_ref[2] = sz


def _run_fps(coords_p, far0):
    return pl.pallas_call(
        _fps_body,
        out_shape=(
            jax.ShapeDtypeStruct((3, B, S), jnp.float32),
            jax.ShapeDtypeStruct((B, N), jnp.float32),
        ),
    )(coords_p, far0)


def _rne_bf16(v):
    u = lax.bitcast_convert_type(v, jnp.uint32)
    r = (u + jnp.uint32(0x7FFF) + ((u >> 16) & jnp.uint32(1))) \
        & jnp.uint32(0xFFFF0000)
    return lax.bitcast_convert_type(r, jnp.float32)


def _group_body(coords_hbm, data_hbm, cent_hbm, ssq_hbm, out_hbm,
                cx, cy, cz, dx, dy, dz, ssq, cen,
                cxb, cyb, czb, cenb, idxb, featb):
    wid = lax.axis_index("s") * 2 + lax.axis_index("c")
    b = wid // TPB
    q = wid % TPB

    pltpu.sync_copy(coords_hbm.at[pl.ds((b * 3 + 0) * N, N)], cx)
    pltpu.sync_copy(coords_hbm.at[pl.ds((b * 3 + 1) * N, N)], cy)
    pltpu.sync_copy(coords_hbm.at[pl.ds((b * 3 + 2) * N, N)], cz)
    pltpu.sync_copy(data_hbm.at[pl.ds((b * 3 + 0) * N, N)], dx)
    pltpu.sync_copy(data_hbm.at[pl.ds((b * 3 + 1) * N, N)], dy)
    pltpu.sync_copy(data_hbm.at[pl.ds((b * 3 + 2) * N, N)], dz)
    pltpu.sync_copy(ssq_hbm.at[pl.ds(b * N, N)], ssq)
    for c in range(3):
        pltpu.sync_copy(cent_hbm.at[pl.ds((b * 3 + c) * S + q * SPT, SPT)],
                        cen.at[pl.ds(c * SPT, SPT)])

    def rnd_pts(i, _):
        sl = pl.ds(i * 16, 16)
        cxb[sl] = _rne_bf16(cx[sl])
        cyb[sl] = _rne_bf16(cy[sl])
        czb[sl] = _rne_bf16(cz[sl])
        return 0

    lax.fori_loop(0, N // 16, rnd_pts, 0)

    def rnd_cen(i, _):
        sl = pl.ds(i * 16, 16)
        cenb[sl] = _rne_bf16(cen[sl])
        return 0

    lax.fori_loop(0, (3 * SPT) // 16, rnd_cen, 0)

    def row_body(r, _):
        lane = lax.broadcasted_iota(jnp.int32, (16,), 0)
        sx = cen[pl.ds(0 * SPT + r, 16)][0]
        sy = cen[pl.ds(1 * SPT + r, 16)][0]
        sz = cen[pl.ds(2 * SPT + r, 16)][0]
        sxb = cenb[pl.ds(0 * SPT + r, 16)][0]
        syb = cenb[pl.ds(1 * SPT + r, 16)][0]
        szb = cenb[pl.ds(2 * SPT + r, 16)][0]
        cs2 = (sx * sx + sy * sy) + sz * sz

        def chunk(c, cnt):
            base = c * 16
            xv = cxb[pl.ds(base, 16)]
            yv = cyb[pl.ds(base, 16)]
            zv = czb[pl.ds(base, 16)]
            sv = ssq[pl.ds(base, 16)]
            dot = (sxb * xv + syb * yv) + szb * zv
            sq = (-2.0 * dot + cs2) + sv
            mask = jnp.logical_and(sq <= RADIUS_SQ, cnt < K)
            p = plsc.all_reduce_population_count(mask)[0]

            def do_store(c_):
                key = jnp.where(mask, lane + base, jnp.int32(1 << 20))
                _, srt = plsc.sort_key_val(key, key)
                idxb[pl.ds(jnp.minimum(c_, K), 16)] = srt
                return c_ + p

            return lax.cond(p > 0, do_store, lambda c_: c_, cnt)

        def sup(si, cnt):
            return lax.cond(
                cnt < K,
                lambda c: lax.fori_loop(si * 16, si * 16 + 16, chunk, c),
                lambda c: c,
                cnt)

        cnt = lax.fori_loop(0, N // 256, sup, jnp.int32(0))

        first = idxb[pl.ds(0, 16)][0]
        for j in range(2):
            v = idxb[pl.ds(16 * j, 16)]
            v = jnp.where(lane + 16 * j < cnt, v, first)
            iv = v
            gx = plsc.load_gather(cx, [iv]) - sx
            gy = plsc.load_gather(cy, [iv]) - sy
            gz = plsc.load_gather(cz, [iv]) - sz
            o = r * K + 16 * j
            featb[pl.ds(0 * CPT + o, 16)] = gx
            featb[pl.ds(1 * CPT + o, 16)] = gy
            featb[pl.ds(2 * CPT + o, 16)] = gz
            featb[pl.ds(3 * CPT + o, 16)] = plsc.load_gather(dx, [iv])
            featb[pl.ds(4 * CPT + o, 16)] = plsc.load_gather(dy, [iv])
            featb[pl.ds(5 * CPT + o, 16)] = plsc.load_gather(dz, [iv])
        return 0

    lax.fori_loop(0, SPT, row_body, 0)
    for ch in range(6):
        pltpu.sync_copy(featb.at[pl.ds(ch * CPT, CPT)],
                        out_hbm.at[pl.ds(ch * M + wid * CPT, CPT)])


def _run_group(coords_p, data_p, cent, ssq):
    mesh = plsc.VectorSubcoreMesh(core_axis_name="c", subcore_axis_name="s")
    f = functools.partial(
        pl.kernel,
        mesh=mesh,
        compiler_params=pltpu.CompilerParams(needs_layout_passes=False),
        out_type=jax.ShapeDtypeStruct((6 * M,), jnp.float32),
        scratch_types=[
            pltpu.VMEM((N,), jnp.float32),
            pltpu.VMEM((N,), jnp.float32),
            pltpu.VMEM((N,), jnp.float32),
            pltpu.VMEM((N,), jnp.float32),
            pltpu.VMEM((N,), jnp.float32),
            pltpu.VMEM((N,), jnp.float32),
            pltpu.VMEM((N,), jnp.float32),
            pltpu.VMEM((3 * SPT + 16,), jnp.float32),
            pltpu.VMEM((N,), jnp.float32),
            pltpu.VMEM((N,), jnp.float32),
            pltpu.VMEM((N,), jnp.float32),
            pltpu.VMEM((3 * SPT + 16,), jnp.float32),
            pltpu.VMEM((K + 16,), jnp.int32),
            pltpu.VMEM((6 * CPT,), jnp.float32),
        ],
    )(_group_body)
    out = f(coords_p.reshape(-1), data_p.reshape(-1), cent.reshape(-1),
            ssq.reshape(-1))
    return out.reshape(6, M)


MLP_BLK = 4096
NF = float(M)


def _l1_body(x_ref, w_ref, b_ref, y_ref, s_ref, ss_ref):
    y = jnp.dot(w_ref[...], x_ref[...],
                preferred_element_type=jnp.float32) + b_ref[...]
    y_ref[...] = y

    @pl.when(pl.program_id(0) == 0)
    def _():
        s_ref[...] = jnp.zeros_like(s_ref)
        ss_ref[...] = jnp.zeros_like(ss_ref)

    s_ref[...] += jnp.sum(y, axis=1, keepdims=True)
    ss_ref[...] += jnp.sum(y * y, axis=1, keepdims=True)


def _lmid_body(x_ref, s_ref, ss_ref, g_ref, be_ref, w_ref, b_ref,
               y_ref, s2_ref, ss2_ref):
    mean = s_ref[...] / NF
    var = ss_ref[...] / NF - mean * mean
    scale = g_ref[...] / jnp.sqrt(var + 1e-5)
    shift = be_ref[...] - mean * scale
    xn = jnp.maximum(x_ref[...] * scale + shift, 0.0)
    y = jnp.dot(w_ref[...], xn,
                preferred_element_type=jnp.float32) + b_ref[...]
    y_ref[...] = y

    @pl.when(pl.program_id(0) == 0)
    def _():
        s2_ref[...] = jnp.zeros_like(s2_ref)
        ss2_ref[...] = jnp.zeros_like(ss2_ref)

    s2_ref[...] += jnp.sum(y, axis=1, keepdims=True)
    ss2_ref[...] += jnp.sum(y * y, axis=1, keepdims=True)


def _l4_body(x_ref, s_ref, ss_ref, g_ref, be_ref, out_ref):
    mean = s_ref[...] / NF
    var = ss_ref[...] / NF - mean * mean
    scale = (g_ref[...] / jnp.sqrt(var + 1e-5))[:, :, None]
    shift = (be_ref[...] - (s_ref[...] / NF) * scale[:, :, 0])[:, :, None]
    xn = jnp.maximum(x_ref[...] * scale + shift, 0.0)
    out_ref[...] = jnp.max(xn, axis=-1)


def _run_mlp(feats, W0, b0, g0, be0, W1, b1, g1, be1, W2, b2, g2, be2):
    col = lambda v: v.reshape(-1, 1)
    nblk = M // MLP_BLK
    x1, s1, ss1 = pl.pallas_call(
        _l1_body,
        grid=(nblk,),
        in_specs=[
            pl.BlockSpec((6, MLP_BLK), lambda i: (0, i)),
            pl.BlockSpec((32, 6), lambda i: (0, 0)),
            pl.BlockSpec((32, 1), lambda i: (0, 0)),
        ],
        out_specs=(
            pl.BlockSpec((32, MLP_BLK), lambda i: (0, i)),
            pl.BlockSpec((32, 1), lambda i: (0, 0)),
            pl.BlockSpec((32, 1), lambda i: (0, 0)),
        ),
        out_shape=(
            jax.ShapeDtypeStruct((32, M), jnp.float32),
            jax.ShapeDtypeStruct((32, 1), jnp.float32),
            jax.ShapeDtypeStruct((32, 1), jnp.float32),
        ),
    )(feats, W0, col(b0))

    def mid(x, s, ss, g, be, Wn, bn, cin, cout):
        return pl.pallas_call(
            _lmid_body,
            grid=(nblk,),
            in_specs=[
                pl.BlockSpec((cin, MLP_BLK), lambda i: (0, i)),
                pl.BlockSpec((cin, 1), lambda i: (0, 0)),
                pl.BlockSpec((cin, 1), lambda i: (0, 0)),
                pl.BlockSpec((cin, 1), lambda i: (0, 0)),
                pl.BlockSpec((cin, 1), lambda i: (0, 0)),
                pl.BlockSpec((cout, cin), lambda i: (0, 0)),
                pl.BlockSpec((cout, 1), lambda i: (0, 0)),
            ],
            out_specs=(
                pl.BlockSpec((cout, MLP_BLK), lambda i: (0, i)),
                pl.BlockSpec((cout, 1), lambda i: (0, 0)),
                pl.BlockSpec((cout, 1), lambda i: (0, 0)),
            ),
            out_shape=(
                jax.ShapeDtypeStruct((cout, M), jnp.float32),
                jax.ShapeDtypeStruct((cout, 1), jnp.float32),
                jax.ShapeDtypeStruct((cout, 1), jnp.float32),
            ),
        )(x, s, ss, col(g), col(be), Wn, col(bn))

    x2, s2, ss2 = mid(x1, s1, ss1, g0, be0, W1, b1, 32, 32)
    x3, s3, ss3 = mid(x2, s2, ss2, g1, be1, W2, b2, 32, 64)

    x3v = x3.reshape(64, B * S, K)
    GBLK = 128
    feats_out = pl.pallas_call(
        _l4_body,
        grid=(B * S // GBLK,),
        in_specs=[
            pl.BlockSpec((64, GBLK, K), lambda i: (0, i, 0)),
            pl.BlockSpec((64, 1), lambda i: (0, 0)),
            pl.BlockSpec((64, 1), lambda i: (0, 0)),
            pl.BlockSpec((64, 1), lambda i: (0, 0)),
            pl.BlockSpec((64, 1), lambda i: (0, 0)),
        ],
        out_specs=pl.BlockSpec((64, GBLK), lambda i: (0, i)),
        out_shape=jax.ShapeDtypeStruct((64, B * S), jnp.float32),
    )(x3v, s3, ss3, col(g2), col(be2))
    return feats_out


def kernel(coords, data, W0, b0, g0, be0, W1, b1, g1, be1, W2, b2, g2, be2):
    coords_p = jnp.transpose(coords, (0, 2, 1))
    data_p = jnp.transpose(data, (0, 2, 1))
    far0 = jax.random.randint(jax.random.key(1), (B,), 0, N)
    far0 = far0.astype(jnp.int32).reshape(B, 1)

    scout, ssq = _run_fps(jnp.transpose(coords_p, (1, 0, 2)), far0)
    sample_coords = jnp.transpose(scout, (1, 2, 0))
    cent = jnp.transpose(scout, (1, 0, 2))

    feats6 = _run_group(coords_p, data_p, cent, ssq)

    fo = _run_mlp(feats6, W0, b0, g0, be0, W1, b1, g1, be1,
                  W2, b2, g2, be2)
    sample_feats = jnp.transpose(fo, (1, 0)).reshape(B, S, 64)
    return sample_coords, sample_feats


# speedup vs baseline: 1.5792x; 1.5792x over previous
import functools

import jax
import jax.numpy as jnp
from jax import lax
from jax.experimental import pallas as pl
from jax.experimental.pallas import tpu as pltpu
from jax.experimental.pallas import tpu_sc as plsc

B = 8
N = 4096
S = 512
K = 32
M = B * S * K
RADIUS_SQ = 1.0
NTILES = 32
SPT = S // (NTILES // B)
TPB = NTILES // B
CPT = SPT * K


def _fps_body(cp_ref, far0_ref, scout_ref, ssq_ref):
    x = cp_ref[0]
    y = cp_ref[1]
    z = cp_ref[2]
    ssq_ref[...] = (x * x + y * y) + z * z
    iota = lax.broadcasted_iota(jnp.int32, (B, N), 1)
    iota_s = lax.broadcasted_iota(jnp.int32, (B, S), 1)

    def body(i, carry):
        dist, far, sx, sy, sz = carry
        oh = iota == far
        cxs = jnp.sum(jnp.where(oh, x, 0.0), axis=1, keepdims=True)
        cys = jnp.sum(jnp.where(oh, y, 0.0), axis=1, keepdims=True)
        czs = jnp.sum(jnp.where(oh, z, 0.0), axis=1, keepdims=True)
        hit = iota_s == i
        sx = jnp.where(hit, cxs, sx)
        sy = jnp.where(hit, cys, sy)
        sz = jnp.where(hit, czs, sz)
        dx = x - cxs
        dy = y - cys
        dz = z - czs
        d = (dx * dx + dy * dy) + dz * dz
        dist = jnp.minimum(dist, d)
        m = jnp.max(dist, axis=1, keepdims=True)
        far = jnp.min(jnp.where(dist == m, iota, jnp.int32(N)), axis=1,
                      keepdims=True)
        return dist, far, sx, sy, sz

    dist0 = jnp.full((B, N), jnp.inf, dtype=jnp.float32)
    z0 = jnp.zeros((B, S), dtype=jnp.float32)
    _, _, sx, sy, sz = lax.fori_loop(
        0, S, body, (dist0, far0_ref[...], z0, z0, z0))
    scout_ref[0] = sx
    scout_ref[1] = sy
    scout_ref[2] = sz


def _run_fps(coords_p, far0):
    return pl.pallas_call(
        _fps_body,
        out_shape=(
            jax.ShapeDtypeStruct((3, B, S), jnp.float32),
            jax.ShapeDtypeStruct((B, N), jnp.float32),
        ),
    )(coords_p, far0)


def _rne_bf16(v):
    u = lax.bitcast_convert_type(v, jnp.uint32)
    r = (u + jnp.uint32(0x7FFF) + ((u >> 16) & jnp.uint32(1))) \
        & jnp.uint32(0xFFFF0000)
    return lax.bitcast_convert_type(r, jnp.float32)


def _group_body(coords_hbm, data_hbm, cent_hbm, ssq_hbm, out_hbm,
                cx, cy, cz, dx, dy, dz, ssq, cen,
                cxb, cyb, czb, cenb, idxb, featb):
    wid = lax.axis_index("s") * 2 + lax.axis_index("c")
    b = wid // TPB
    q = wid % TPB

    pltpu.sync_copy(coords_hbm.at[pl.ds((b * 3 + 0) * N, N)], cx)
    pltpu.sync_copy(coords_hbm.at[pl.ds((b * 3 + 1) * N, N)], cy)
    pltpu.sync_copy(coords_hbm.at[pl.ds((b * 3 + 2) * N, N)], cz)
    pltpu.sync_copy(data_hbm.at[pl.ds((b * 3 + 0) * N, N)], dx)
    pltpu.sync_copy(data_hbm.at[pl.ds((b * 3 + 1) * N, N)], dy)
    pltpu.sync_copy(data_hbm.at[pl.ds((b * 3 + 2) * N, N)], dz)
    pltpu.sync_copy(ssq_hbm.at[pl.ds(b * N, N)], ssq)
    for c in range(3):
        pltpu.sync_copy(cent_hbm.at[pl.ds((b * 3 + c) * S + q * SPT, SPT)],
                        cen.at[pl.ds(c * SPT, SPT)])

    def rnd_pts(i, _):
        sl = pl.ds(i * 16, 16)
        cxb[sl] = _rne_bf16(cx[sl])
        cyb[sl] = _rne_bf16(cy[sl])
        czb[sl] = _rne_bf16(cz[sl])
        return 0

    lax.fori_loop(0, N // 16, rnd_pts, 0)

    def rnd_cen(i, _):
        sl = pl.ds(i * 16, 16)
        cenb[sl] = _rne_bf16(cen[sl])
        return 0

    lax.fori_loop(0, (3 * SPT) // 16, rnd_cen, 0)

    def row_body(r, _):
        lane = lax.broadcasted_iota(jnp.int32, (16,), 0)
        sx = cen[pl.ds(0 * SPT + r, 16)][0]
        sy = cen[pl.ds(1 * SPT + r, 16)][0]
        sz = cen[pl.ds(2 * SPT + r, 16)][0]
        sxb = cenb[pl.ds(0 * SPT + r, 16)][0]
        syb = cenb[pl.ds(1 * SPT + r, 16)][0]
        szb = cenb[pl.ds(2 * SPT + r, 16)][0]
        cs2 = (sx * sx + sy * sy) + sz * sz

        UNROLL = 2

        def chunk(c, cnt):
            srts = []
            pops = []
            for t in range(UNROLL):
                base = c * (16 * UNROLL) + t * 16
                xv = cxb[pl.ds(base, 16)]
                yv = cyb[pl.ds(base, 16)]
                zv = czb[pl.ds(base, 16)]
                sv = ssq[pl.ds(base, 16)]
                dot = (sxb * xv + syb * yv) + szb * zv
                sq = (-2.0 * dot + cs2) + sv
                mask = sq <= RADIUS_SQ
                key = jnp.where(mask, lane + base, jnp.int32(1 << 20))
                _, srt = plsc.sort_key_val(key, key)
                srts.append(srt)
                pops.append(plsc.all_reduce_population_count(mask)[0])
            for t in range(UNROLL):
                live = cnt < K
                idxb[pl.ds(jnp.minimum(cnt, K), 16)] = jnp.where(
                    live, srts[t], jnp.int32(1 << 20))
                cnt = cnt + jnp.where(live, pops[t], 0)
            return cnt

        def sup(si, cnt):
            per = 16 // UNROLL
            return lax.cond(
                cnt < K,
                lambda c: lax.fori_loop(si * per, si * per + per, chunk, c),
                lambda c: c,
                cnt)

        cnt = lax.fori_loop(0, N // 256, sup, jnp.int32(0))

        first = idxb[pl.ds(0, 16)][0]
        for j in range(2):
            v = idxb[pl.ds(16 * j, 16)]
            v = jnp.where(lane + 16 * j < cnt, v, first)
            iv = v
            gx = plsc.load_gather(cx, [iv]) - sx
            gy = plsc.load_gather(cy, [iv]) - sy
            gz = plsc.load_gather(cz, [iv]) - sz
            o = r * K + 16 * j
            featb[pl.ds(0 * CPT + o, 16)] = gx
            featb[pl.ds(1 * CPT + o, 16)] = gy
            featb[pl.ds(2 * CPT + o, 16)] = gz
            featb[pl.ds(3 * CPT + o, 16)] = plsc.load_gather(dx, [iv])
            featb[pl.ds(4 * CPT + o, 16)] = plsc.load_gather(dy, [iv])
            featb[pl.ds(5 * CPT + o, 16)] = plsc.load_gather(dz, [iv])
        return 0

    lax.fori_loop(0, SPT, row_body, 0)
    for ch in range(6):
        pltpu.sync_copy(featb.at[pl.ds(ch * CPT, CPT)],
                        out_hbm.at[pl.ds(ch * M + wid * CPT, CPT)])


def _run_group(coords_p, data_p, cent, ssq):
    mesh = plsc.VectorSubcoreMesh(core_axis_name="c", subcore_axis_name="s")
    f = functools.partial(
        pl.kernel,
        mesh=mesh,
        compiler_params=pltpu.CompilerParams(needs_layout_passes=False),
        out_type=jax.ShapeDtypeStruct((6 * M,), jnp.float32),
        scratch_types=[
            pltpu.VMEM((N,), jnp.float32),
            pltpu.VMEM((N,), jnp.float32),
            pltpu.VMEM((N,), jnp.float32),
            pltpu.VMEM((N,), jnp.float32),
            pltpu.VMEM((N,), jnp.float32),
            pltpu.VMEM((N,), jnp.float32),
            pltpu.VMEM((N,), jnp.float32),
            pltpu.VMEM((3 * SPT + 16,), jnp.float32),
            pltpu.VMEM((N,), jnp.float32),
            pltpu.VMEM((N,), jnp.float32),
            pltpu.VMEM((N,), jnp.float32),
            pltpu.VMEM((3 * SPT + 16,), jnp.float32),
            pltpu.VMEM((K + 16,), jnp.int32),
            pltpu.VMEM((6 * CPT,), jnp.float32),
        ],
    )(_group_body)
    out = f(coords_p.reshape(-1), data_p.reshape(-1), cent.reshape(-1),
            ssq.reshape(-1))
    return out.reshape(6, M)


MLP_BLK = 4096
NF = float(M)


def _l1_body(x_ref, w_ref, b_ref, y_ref, s_ref, ss_ref):
    y = jnp.dot(w_ref[...], x_ref[...],
                preferred_element_type=jnp.float32) + b_ref[...]
    y_ref[...] = y

    @pl.when(pl.program_id(0) == 0)
    def _():
        s_ref[...] = jnp.zeros_like(s_ref)
        ss_ref[...] = jnp.zeros_like(ss_ref)

    s_ref[...] += jnp.sum(y, axis=1, keepdims=True)
    ss_ref[...] += jnp.sum(y * y, axis=1, keepdims=True)


def _lmid_body(x_ref, s_ref, ss_ref, g_ref, be_ref, w_ref, b_ref,
               y_ref, s2_ref, ss2_ref):
    mean = s_ref[...] / NF
    var = ss_ref[...] / NF - mean * mean
    scale = g_ref[...] / jnp.sqrt(var + 1e-5)
    shift = be_ref[...] - mean * scale
    xn = jnp.maximum(x_ref[...] * scale + shift, 0.0)
    y = jnp.dot(w_ref[...], xn,
                preferred_element_type=jnp.float32) + b_ref[...]
    y_ref[...] = y

    @pl.when(pl.program_id(0) == 0)
    def _():
        s2_ref[...] = jnp.zeros_like(s2_ref)
        ss2_ref[...] = jnp.zeros_like(ss2_ref)

    s2_ref[...] += jnp.sum(y, axis=1, keepdims=True)
    ss2_ref[...] += jnp.sum(y * y, axis=1, keepdims=True)


def _l4_body(x_ref, s_ref, ss_ref, g_ref, be_ref, out_ref):
    mean = s_ref[...] / NF
    var = ss_ref[...] / NF - mean * mean
    scale = (g_ref[...] / jnp.sqrt(var + 1e-5))[:, :, None]
    shift = (be_ref[...] - (s_ref[...] / NF) * scale[:, :, 0])[:, :, None]
    xn = jnp.maximum(x_ref[...] * scale + shift, 0.0)
    out_ref[...] = jnp.max(xn, axis=-1)


def _run_mlp(feats, W0, b0, g0, be0, W1, b1, g1, be1, W2, b2, g2, be2):
    col = lambda v: v.reshape(-1, 1)
    nblk = M // MLP_BLK
    x1, s1, ss1 = pl.pallas_call(
        _l1_body,
        grid=(nblk,),
        in_specs=[
            pl.BlockSpec((6, MLP_BLK), lambda i: (0, i)),
            pl.BlockSpec((32, 6), lambda i: (0, 0)),
            pl.BlockSpec((32, 1), lambda i: (0, 0)),
        ],
        out_specs=(
            pl.BlockSpec((32, MLP_BLK), lambda i: (0, i)),
            pl.BlockSpec((32, 1), lambda i: (0, 0)),
            pl.BlockSpec((32, 1), lambda i: (0, 0)),
        ),
        out_shape=(
            jax.ShapeDtypeStruct((32, M), jnp.float32),
            jax.ShapeDtypeStruct((32, 1), jnp.float32),
            jax.ShapeDtypeStruct((32, 1), jnp.float32),
        ),
    )(feats, W0, col(b0))

    def mid(x, s, ss, g, be, Wn, bn, cin, cout):
        return pl.pallas_call(
            _lmid_body,
            grid=(nblk,),
            in_specs=[
                pl.BlockSpec((cin, MLP_BLK), lambda i: (0, i)),
                pl.BlockSpec((cin, 1), lambda i: (0, 0)),
                pl.BlockSpec((cin, 1), lambda i: (0, 0)),
                pl.BlockSpec((cin, 1), lambda i: (0, 0)),
                pl.BlockSpec((cin, 1), lambda i: (0, 0)),
                pl.BlockSpec((cout, cin), lambda i: (0, 0)),
                pl.BlockSpec((cout, 1), lambda i: (0, 0)),
            ],
            out_specs=(
                pl.BlockSpec((cout, MLP_BLK), lambda i: (0, i)),
                pl.BlockSpec((cout, 1), lambda i: (0, 0)),
                pl.BlockSpec((cout, 1), lambda i: (0, 0)),
            ),
            out_shape=(
                jax.ShapeDtypeStruct((cout, M), jnp.float32),
                jax.ShapeDtypeStruct((cout, 1), jnp.float32),
                jax.ShapeDtypeStruct((cout, 1), jnp.float32),
            ),
        )(x, s, ss, col(g), col(be), Wn, col(bn))

    x2, s2, ss2 = mid(x1, s1, ss1, g0, be0, W1, b1, 32, 32)
    x3, s3, ss3 = mid(x2, s2, ss2, g1, be1, W2, b2, 32, 64)

    x3v = x3.reshape(64, B * S, K)
    GBLK = 128
    feats_out = pl.pallas_call(
        _l4_body,
        grid=(B * S // GBLK,),
        in_specs=[
            pl.BlockSpec((64, GBLK, K), lambda i: (0, i, 0)),
            pl.BlockSpec((64, 1), lambda i: (0, 0)),
            pl.BlockSpec((64, 1), lambda i: (0, 0)),
            pl.BlockSpec((64, 1), lambda i: (0, 0)),
            pl.BlockSpec((64, 1), lambda i: (0, 0)),
        ],
        out_specs=pl.BlockSpec((64, GBLK), lambda i: (0, i)),
        out_shape=jax.ShapeDtypeStruct((64, B * S), jnp.float32),
    )(x3v, s3, ss3, col(g2), col(be2))
    return feats_out


def kernel(coords, data, W0, b0, g0, be0, W1, b1, g1, be1, W2, b2, g2, be2):
    coords_p = jnp.transpose(coords, (0, 2, 1))
    data_p = jnp.transpose(data, (0, 2, 1))
    far0 = jax.random.randint(jax.random.key(1), (B,), 0, N)
    far0 = far0.astype(jnp.int32).reshape(B, 1)

    scout, ssq = _run_fps(jnp.transpose(coords_p, (1, 0, 2)), far0)
    sample_coords = jnp.transpose(scout, (1, 2, 0))
    cent = jnp.transpose(scout, (1, 0, 2))

    feats6 = _run_group(coords_p, data_p, cent, ssq)

    fo = _run_mlp(feats6, W0, b0, g0, be0, W1, b1, g1, be1,
                  W2, b2, g2, be2)
    sample_feats = jnp.transpose(fo, (1, 0)).reshape(B, S, 64)
    return sample_coords, sample_feats


# speedup vs baseline: 1.8548x; 1.1745x over previous
import functools

import jax
import jax.numpy as jnp
from jax import lax
from jax.experimental import pallas as pl
from jax.experimental.pallas import tpu as pltpu
from jax.experimental.pallas import tpu_sc as plsc

B = 8
N = 4096
S = 512
K = 32
M = B * S * K
RADIUS_SQ = 1.0
NTILES = 32
SPT = S // (NTILES // B)
TPB = NTILES // B
CPT = SPT * K


def _fps_body(cp_ref, far0_ref, scout_ref, ssq_ref):
    x = cp_ref[0]
    y = cp_ref[1]
    z = cp_ref[2]
    ssq_ref[...] = (x * x + y * y) + z * z
    iota = lax.broadcasted_iota(jnp.int32, (B, N), 1)
    iota_s = lax.broadcasted_iota(jnp.int32, (B, S), 1)

    def body(i, carry):
        dist, far, sx, sy, sz = carry
        oh = iota == far
        cxs = jnp.sum(jnp.where(oh, x, 0.0), axis=1, keepdims=True)
        cys = jnp.sum(jnp.where(oh, y, 0.0), axis=1, keepdims=True)
        czs = jnp.sum(jnp.where(oh, z, 0.0), axis=1, keepdims=True)
        hit = iota_s == i
        sx = jnp.where(hit, cxs, sx)
        sy = jnp.where(hit, cys, sy)
        sz = jnp.where(hit, czs, sz)
        dx = x - cxs
        dy = y - cys
        dz = z - czs
        d = (dx * dx + dy * dy) + dz * dz
        dist = jnp.minimum(dist, d)
        m = jnp.max(dist, axis=1, keepdims=True)
        far = jnp.min(jnp.where(dist == m, iota, jnp.int32(N)), axis=1,
                      keepdims=True)
        return dist, far, sx, sy, sz

    dist0 = jnp.full((B, N), jnp.inf, dtype=jnp.float32)
    z0 = jnp.zeros((B, S), dtype=jnp.float32)
    _, _, sx, sy, sz = lax.fori_loop(
        0, S, body, (dist0, far0_ref[...], z0, z0, z0))
    scout_ref[0] = sx
    scout_ref[1] = sy
    scout_ref[2] = sz


def _run_fps(coords_p, far0):
    return pl.pallas_call(
        _fps_body,
        out_shape=(
            jax.ShapeDtypeStruct((3, B, S), jnp.float32),
            jax.ShapeDtypeStruct((B, N), jnp.float32),
        ),
    )(coords_p, far0)


def _rne_bf16(v):
    u = lax.bitcast_convert_type(v, jnp.uint32)
    r = (u + jnp.uint32(0x7FFF) + ((u >> 16) & jnp.uint32(1))) \
        & jnp.uint32(0xFFFF0000)
    return lax.bitcast_convert_type(r, jnp.float32)


def _group_body(coords_hbm, data_hbm, cent_hbm, ssq_hbm, out_hbm,
                cx, cy, cz, dx, dy, dz, ssq, cen,
                cxb, cyb, czb, cenb, idxb, featb):
    wid = lax.axis_index("s") * 2 + lax.axis_index("c")
    b = wid // TPB
    q = wid % TPB

    pltpu.sync_copy(coords_hbm.at[pl.ds((b * 3 + 0) * N, N)], cx)
    pltpu.sync_copy(coords_hbm.at[pl.ds((b * 3 + 1) * N, N)], cy)
    pltpu.sync_copy(coords_hbm.at[pl.ds((b * 3 + 2) * N, N)], cz)
    pltpu.sync_copy(data_hbm.at[pl.ds((b * 3 + 0) * N, N)], dx)
    pltpu.sync_copy(data_hbm.at[pl.ds((b * 3 + 1) * N, N)], dy)
    pltpu.sync_copy(data_hbm.at[pl.ds((b * 3 + 2) * N, N)], dz)
    pltpu.sync_copy(ssq_hbm.at[pl.ds(b * N, N)], ssq)
    for c in range(3):
        pltpu.sync_copy(cent_hbm.at[pl.ds((b * 3 + c) * S + q * SPT, SPT)],
                        cen.at[pl.ds(c * SPT, SPT)])

    def rnd_pts(i, _):
        sl = pl.ds(i * 16, 16)
        cxb[sl] = _rne_bf16(cx[sl])
        cyb[sl] = _rne_bf16(cy[sl])
        czb[sl] = _rne_bf16(cz[sl])
        return 0

    lax.fori_loop(0, N // 16, rnd_pts, 0)

    def rnd_cen(i, _):
        sl = pl.ds(i * 16, 16)
        cenb[sl] = _rne_bf16(cen[sl])
        return 0

    lax.fori_loop(0, (3 * SPT) // 16, rnd_cen, 0)

    def row_body(r, _):
        lane = lax.broadcasted_iota(jnp.int32, (16,), 0)
        sx = cen[pl.ds(0 * SPT + r, 16)][0]
        sy = cen[pl.ds(1 * SPT + r, 16)][0]
        sz = cen[pl.ds(2 * SPT + r, 16)][0]
        sxb = cenb[pl.ds(0 * SPT + r, 16)][0]
        syb = cenb[pl.ds(1 * SPT + r, 16)][0]
        szb = cenb[pl.ds(2 * SPT + r, 16)][0]
        cs2 = (sx * sx + sy * sy) + sz * sz

        UNROLL = 4

        def chunk(c, cnt):
            srts = []
            pops = []
            for t in range(UNROLL):
                base = c * (16 * UNROLL) + t * 16
                xv = cxb[pl.ds(base, 16)]
                yv = cyb[pl.ds(base, 16)]
                zv = czb[pl.ds(base, 16)]
                sv = ssq[pl.ds(base, 16)]
                dot = (sxb * xv + syb * yv) + szb * zv
                sq = (-2.0 * dot + cs2) + sv
                mask = sq <= RADIUS_SQ
                key = jnp.where(mask, lane + base, jnp.int32(1 << 20))
                _, srt = plsc.sort_key_val(key, key)
                srts.append(srt)
                pops.append(plsc.all_reduce_population_count(mask)[0])
            for t in range(UNROLL):
                live = cnt < K
                idxb[pl.ds(jnp.minimum(cnt, K), 16)] = jnp.where(
                    live, srts[t], jnp.int32(1 << 20))
                cnt = cnt + jnp.where(live, pops[t], 0)
            return cnt

        def sup(si, cnt):
            per = 16 // UNROLL
            return lax.cond(
                cnt < K,
                lambda c: lax.fori_loop(si * per, si * per + per, chunk, c),
                lambda c: c,
                cnt)

        cnt = lax.fori_loop(0, N // 256, sup, jnp.int32(0))

        first = idxb[pl.ds(0, 16)][0]
        for j in range(2):
            v = idxb[pl.ds(16 * j, 16)]
            v = jnp.where(lane + 16 * j < cnt, v, first)
            iv = v
            gx = plsc.load_gather(cx, [iv]) - sx
            gy = plsc.load_gather(cy, [iv]) - sy
            gz = plsc.load_gather(cz, [iv]) - sz
            o = r * K + 16 * j
            featb[pl.ds(0 * CPT + o, 16)] = gx
            featb[pl.ds(1 * CPT + o, 16)] = gy
            featb[pl.ds(2 * CPT + o, 16)] = gz
            featb[pl.ds(3 * CPT + o, 16)] = plsc.load_gather(dx, [iv])
            featb[pl.ds(4 * CPT + o, 16)] = plsc.load_gather(dy, [iv])
            featb[pl.ds(5 * CPT + o, 16)] = plsc.load_gather(dz, [iv])
        return 0

    lax.fori_loop(0, SPT, row_body, 0)
    for ch in range(6):
        pltpu.sync_copy(featb.at[pl.ds(ch * CPT, CPT)],
                        out_hbm.at[pl.ds(ch * M + wid * CPT, CPT)])


def _run_group(coords_p, data_p, cent, ssq):
    mesh = plsc.VectorSubcoreMesh(core_axis_name="c", subcore_axis_name="s")
    f = functools.partial(
        pl.kernel,
        mesh=mesh,
        compiler_params=pltpu.CompilerParams(needs_layout_passes=False),
        out_type=jax.ShapeDtypeStruct((6 * M,), jnp.float32),
        scratch_types=[
            pltpu.VMEM((N,), jnp.float32),
            pltpu.VMEM((N,), jnp.float32),
            pltpu.VMEM((N,), jnp.float32),
            pltpu.VMEM((N,), jnp.float32),
            pltpu.VMEM((N,), jnp.float32),
            pltpu.VMEM((N,), jnp.float32),
            pltpu.VMEM((N,), jnp.float32),
            pltpu.VMEM((3 * SPT + 16,), jnp.float32),
            pltpu.VMEM((N,), jnp.float32),
            pltpu.VMEM((N,), jnp.float32),
            pltpu.VMEM((N,), jnp.float32),
            pltpu.VMEM((3 * SPT + 16,), jnp.float32),
            pltpu.VMEM((K + 16,), jnp.int32),
            pltpu.VMEM((6 * CPT,), jnp.float32),
        ],
    )(_group_body)
    out = f(coords_p.reshape(-1), data_p.reshape(-1), cent.reshape(-1),
            ssq.reshape(-1))
    return out.reshape(6, M)


MLP_BLK = 4096
NF = float(M)


def _l4_body(x_ref, s_ref, ss_ref, g_ref, be_ref, out_ref):
    mean = s_ref[...] / NF
    var = ss_ref[...] / NF - mean * mean
    scale = (g_ref[...] / jnp.sqrt(var + 1e-5))[:, :, None]
    shift = (be_ref[...] - (s_ref[...] / NF) * scale[:, :, 0])[:, :, None]
    xn = jnp.maximum(x_ref[...] * scale + shift, 0.0)
    out_ref[...] = jnp.max(xn, axis=-1)


def _mlp3_body(feats_ref, w0_ref, b0_ref, g0_ref, be0_ref,
               w1_ref, b1_ref, g1_ref, be1_ref, w2_ref, b2_ref,
               x3_ref, s3_ref, ss3_ref,
               x1s, x2s, s1s, ss1s, s2s, ss2s):
    p = pl.program_id(0)
    j = pl.program_id(1)
    sl = pl.ds(j * MLP_BLK, MLP_BLK)

    def norm(x, s, ss, g, be):
        mean = s / NF
        var = ss / NF - mean * mean
        scale = g / jnp.sqrt(var + 1e-5)
        shift = be - mean * scale
        return jnp.maximum(x * scale + shift, 0.0)

    @pl.when(jnp.logical_and(p == 0, j == 0))
    def _():
        s1s[...] = jnp.zeros_like(s1s)
        ss1s[...] = jnp.zeros_like(ss1s)

    @pl.when(p == 0)
    def _():
        y = jnp.dot(w0_ref[...], feats_ref[...],
                    preferred_element_type=jnp.float32) + b0_ref[...]
        x1s[:, sl] = y
        s1s[...] += jnp.sum(y, axis=1, keepdims=True)
        ss1s[...] += jnp.sum(y * y, axis=1, keepdims=True)

    @pl.when(jnp.logical_and(p == 1, j == 0))
    def _():
        s2s[...] = jnp.zeros_like(s2s)
        ss2s[...] = jnp.zeros_like(ss2s)

    @pl.when(p == 1)
    def _():
        xn = norm(x1s[:, sl], s1s[...], ss1s[...], g0_ref[...], be0_ref[...])
        y = jnp.dot(w1_ref[...], xn,
                    preferred_element_type=jnp.float32) + b1_ref[...]
        x2s[:, sl] = y
        s2s[...] += jnp.sum(y, axis=1, keepdims=True)
        ss2s[...] += jnp.sum(y * y, axis=1, keepdims=True)

    @pl.when(jnp.logical_and(p == 2, j == 0))
    def _():
        s3_ref[...] = jnp.zeros_like(s3_ref)
        ss3_ref[...] = jnp.zeros_like(ss3_ref)

    @pl.when(p == 2)
    def _():
        xn = norm(x2s[:, sl], s2s[...], ss2s[...], g1_ref[...], be1_ref[...])
        y = jnp.dot(w2_ref[...], xn,
                    preferred_element_type=jnp.float32) + b2_ref[...]
        x3_ref[...] = y
        s3_ref[...] += jnp.sum(y, axis=1, keepdims=True)
        ss3_ref[...] += jnp.sum(y * y, axis=1, keepdims=True)


def _run_mlp(feats, W0, b0, g0, be0, W1, b1, g1, be1, W2, b2, g2, be2):
    col = lambda v: v.reshape(-1, 1)
    nblk = M // MLP_BLK
    small = lambda c: pl.BlockSpec((c, 1), lambda p, j: (0, 0))
    x3, s3, ss3 = pl.pallas_call(
        _mlp3_body,
        grid=(3, nblk),
        in_specs=[
            pl.BlockSpec((6, MLP_BLK),
                         lambda p, j: (0, jnp.where(p == 0, j, 0))),
            pl.BlockSpec((32, 6), lambda p, j: (0, 0)),
            small(32), small(32), small(32),
            pl.BlockSpec((32, 32), lambda p, j: (0, 0)),
            small(32), small(32), small(32),
            pl.BlockSpec((64, 32), lambda p, j: (0, 0)),
            small(64),
        ],
        out_specs=(
            pl.BlockSpec((64, MLP_BLK),
                         lambda p, j: (0, jnp.where(p == 2, j, 0))),
            pl.BlockSpec((64, 1), lambda p, j: (0, 0)),
            pl.BlockSpec((64, 1), lambda p, j: (0, 0)),
        ),
        out_shape=(
            jax.ShapeDtypeStruct((64, M), jnp.float32),
            jax.ShapeDtypeStruct((64, 1), jnp.float32),
            jax.ShapeDtypeStruct((64, 1), jnp.float32),
        ),
        scratch_shapes=[
            pltpu.VMEM((32, M), jnp.float32),
            pltpu.VMEM((32, M), jnp.float32),
            pltpu.VMEM((32, 1), jnp.float32),
            pltpu.VMEM((32, 1), jnp.float32),
            pltpu.VMEM((32, 1), jnp.float32),
            pltpu.VMEM((32, 1), jnp.float32),
        ],
    )(feats, W0, col(b0), col(g0), col(be0), W1, col(b1), col(g1), col(be1),
      W2, col(b2))

    x3v = x3.reshape(64, B * S, K)
    GBLK = 128
    feats_out = pl.pallas_call(
        _l4_body,
        grid=(B * S // GBLK,),
        in_specs=[
            pl.BlockSpec((64, GBLK, K), lambda i: (0, i, 0)),
            pl.BlockSpec((64, 1), lambda i: (0, 0)),
            pl.BlockSpec((64, 1), lambda i: (0, 0)),
            pl.BlockSpec((64, 1), lambda i: (0, 0)),
            pl.BlockSpec((64, 1), lambda i: (0, 0)),
        ],
        out_specs=pl.BlockSpec((64, GBLK), lambda i: (0, i)),
        out_shape=jax.ShapeDtypeStruct((64, B * S), jnp.float32),
    )(x3v, s3, ss3, col(g2), col(be2))
    return feats_out


def kernel(coords, data, W0, b0, g0, be0, W1, b1, g1, be1, W2, b2, g2, be2):
    coords_p = jnp.transpose(coords, (0, 2, 1))
    data_p = jnp.transpose(data, (0, 2, 1))
    far0 = jax.random.randint(jax.random.key(1), (B,), 0, N)
    far0 = far0.astype(jnp.int32).reshape(B, 1)

    scout, ssq = _run_fps(jnp.transpose(coords_p, (1, 0, 2)), far0)
    sample_coords = jnp.transpose(scout, (1, 2, 0))
    cent = jnp.transpose(scout, (1, 0, 2))

    feats6 = _run_group(coords_p, data_p, cent, ssq)

    fo = _run_mlp(feats6, W0, b0, g0, be0, W1, b1, g1, be1,
                  W2, b2, g2, be2)
    sample_feats = jnp.transpose(fo, (1, 0)).reshape(B, S, 64)
    return sample_coords, sample_feats
